# baseline (speedup 1.0000x reference)
import functools

import jax
import jax.numpy as jnp
from jax import lax
from jax.experimental import pallas as pl
from jax.experimental.pallas import tpu as pltpu
from jax.experimental.pallas import tpu_sc as plsc

_N = 10000
_E = 320000
_NC = 2
_NS = 16
_NW = _NC * _NS
_NP = 10240
_RPW = _NP // _NS
_R = 1000
_NBUF = 5
_CH = 80
_NCH = _E // (_NW * _CH)


def _sc_mesh():
    return plsc.VectorSubcoreMesh(
        core_axis_name="c", subcore_axis_name="s",
        num_cores=_NC, num_subcores=_NS)


@functools.lru_cache(maxsize=None)
def _edge_scatter(D):
    ch = _CH
    nch = _NCH

    @functools.partial(
        pl.kernel,
        out_type=jax.ShapeDtypeStruct((_NC, _NP, D), jnp.float32),
        mesh=_sc_mesh(),
        compiler_params=pltpu.CompilerParams(use_tc_tiling_on_sc=False),
        scratch_types=[
            pltpu.VMEM((nch, ch), jnp.int32),
            pltpu.VMEM((nch, ch), jnp.int32),
            pltpu.VMEM((_NBUF, ch, D), jnp.float32),
            pltpu.VMEM_SHARED((_NP, D), jnp.float32),
            pltpu.SemaphoreType.DMA,
            pltpu.SemaphoreType.DMA,
        ],
    )
    def k(y_hbm, src_hbm, dst_hbm, out_hbm, src_v, dst_v, rows_v, acc,
          gsem, ssem):
        cid = lax.axis_index("c")
        sid = lax.axis_index("s")
        wid = sid * _NC + cid
        zeros = jnp.zeros((16,), jnp.float32)

        pltpu.async_copy(src_hbm.at[wid], src_v, gsem)
        pltpu.async_copy(dst_hbm.at[wid], dst_v, gsem)

        def zero_body(i, carry):
            for j in range(D // 16):
                rows_v[4, i, 16 * j:16 * (j + 1)] = zeros
            return carry

        lax.fori_loop(0, ch, zero_body, 0)
        pltpu.make_async_copy(src_hbm.at[wid], src_v, gsem).wait()
        pltpu.make_async_copy(dst_hbm.at[wid], dst_v, gsem).wait()

        pltpu.async_copy(y_hbm.at[src_v.at[0]], rows_v.at[0], gsem)
        pltpu.async_copy(y_hbm.at[src_v.at[1]], rows_v.at[1], gsem)
        pltpu.async_copy(y_hbm.at[src_v.at[2]], rows_v.at[2], gsem)
        pltpu.async_copy(y_hbm.at[src_v.at[3]], rows_v.at[3], gsem)

        for t in range(_RPW // ch):
            pltpu.async_copy(
                rows_v.at[4], acc.at[pl.ds(sid * _RPW + t * ch, ch)], ssem)
        for t in range(_RPW // ch):
            pltpu.make_async_copy(
                rows_v.at[4], acc.at[pl.ds(sid * _RPW + t * ch, ch)],
                ssem).wait()
        plsc.subcore_barrier()

        def outer(o, carry):
            for b in range(_NBUF):
                j = _NBUF * o + b
                pltpu.make_async_copy(
                    y_hbm.at[src_v.at[j]], rows_v.at[b], gsem).wait()
                pltpu.async_copy(
                    rows_v.at[b], acc.at[dst_v.at[j]], ssem, add=True)

                @pl.when(j >= 1)
                def _():
                    pltpu.make_async_copy(
                        rows_v.at[(b - 1) % _NBUF], acc.at[pl.ds(0, ch)],
                        ssem).wait()

                @pl.when(j < nch - 4)
                def _():
                    pltpu.async_copy(
                        y_hbm.at[src_v.at[j + 4]],
                        rows_v.at[(b + 4) % _NBUF], gsem)
            return carry

        lax.fori_loop(0, nch // _NBUF, outer, 0)
        pltpu.make_async_copy(
            rows_v.at[0], acc.at[pl.ds(0, ch)], ssem).wait()
        plsc.subcore_barrier()
        pltpu.sync_copy(acc.at[pl.ds(sid * _RPW, _RPW)],
                        out_hbm.at[cid, pl.ds(sid * _RPW, _RPW)])

    return k


@functools.partial(
    pl.kernel,
    out_type=(jax.ShapeDtypeStruct((_NC, _NP, 16), jnp.float32),
              jax.ShapeDtypeStruct((_NC, _NP, 16), jnp.float32)),
    mesh=_sc_mesh(),
    compiler_params=pltpu.CompilerParams(use_tc_tiling_on_sc=False),
    scratch_types=[
        pltpu.VMEM((125, 80), jnp.int32),
        pltpu.VMEM((125, 80), jnp.int32),
        pltpu.VMEM((80, 16), jnp.float32),
        pltpu.VMEM_SHARED((_NP, 16), jnp.float32),
        pltpu.VMEM_SHARED((_NP, 16), jnp.float32),
        pltpu.SemaphoreType.DMA,
        pltpu.SemaphoreType.DMA,
    ],
)
def _deg_kernel(src_hbm, dst_hbm, outs_hbm, outd_hbm,
                src_v, dst_v, ones_v, acc_s, acc_d, gsem, ssem):
    cid = lax.axis_index("c")
    sid = lax.axis_index("s")
    wid = sid * _NC + cid
    zeros = jnp.zeros((16,), jnp.float32)
    ones = jnp.ones((16,), jnp.float32)

    pltpu.async_copy(src_hbm.at[wid], src_v, gsem)
    pltpu.async_copy(dst_hbm.at[wid], dst_v, gsem)

    def zfill(i, carry):
        ones_v[i, 0:16] = zeros
        return carry

    lax.fori_loop(0, 80, zfill, 0)
    for t in range(_RPW // 80):
        pltpu.async_copy(
            ones_v, acc_s.at[pl.ds(sid * _RPW + t * 80, 80)], ssem)
        pltpu.async_copy(
            ones_v, acc_d.at[pl.ds(sid * _RPW + t * 80, 80)], ssem)
    for _ in range(2 * (_RPW // 80)):
        pltpu.make_async_copy(ones_v, acc_s.at[pl.ds(0, 80)], ssem).wait()

    def ofill(i, carry):
        ones_v[i, 0:16] = ones
        return carry

    lax.fori_loop(0, 80, ofill, 0)
    pltpu.make_async_copy(src_hbm.at[wid], src_v, gsem).wait()
    pltpu.make_async_copy(dst_hbm.at[wid], dst_v, gsem).wait()
    plsc.subcore_barrier()

    def body(j, carry):
        pltpu.async_copy(ones_v, acc_s.at[src_v.at[j]], ssem, add=True)
        pltpu.async_copy(ones_v, acc_d.at[dst_v.at[j]], ssem, add=True)

        @pl.when(j >= 8)
        def _():
            pltpu.make_async_copy(
                ones_v, acc_s.at[pl.ds(0, 80)], ssem).wait()
            pltpu.make_async_copy(
                ones_v, acc_s.at[pl.ds(0, 80)], ssem).wait()

        return carry

    lax.fori_loop(0, 125, body, 0)
    for _ in range(16):
        pltpu.make_async_copy(ones_v, acc_s.at[pl.ds(0, 80)], ssem).wait()
    plsc.subcore_barrier()
    pltpu.sync_copy(acc_s.at[pl.ds(sid * _RPW, _RPW)],
                    outs_hbm.at[cid, pl.ds(sid * _RPW, _RPW)])
    pltpu.sync_copy(acc_d.at[pl.ds(sid * _RPW, _RPW)],
                    outd_hbm.at[cid, pl.ds(sid * _RPW, _RPW)])


def _degprep(s_parts, d_parts):
    def kfn(s_ref, d_ref, inv_ref, rso_ref, rsi_ref):
        s = s_ref[0, :, 0:1] + s_ref[1, :, 0:1]
        d = d_ref[0, :, 0:1] + d_ref[1, :, 0:1]
        inv_ref[...] = 1.0 / (d + 2.0)
        rso_ref[...] = lax.rsqrt(s + 1.0)
        rsi_ref[...] = lax.rsqrt(d + 1.0)

    vec = jax.ShapeDtypeStruct((_N, 1), jnp.float32)
    return pl.pallas_call(
        kfn, grid=(_N // _R,),
        in_specs=[pl.BlockSpec((_NC, _R, 16), lambda i: (0, i, 0)),
                  pl.BlockSpec((_NC, _R, 16), lambda i: (0, i, 0))],
        out_specs=[pl.BlockSpec((_R, 1), lambda i: (i, 0))] * 3,
        out_shape=[vec, vec, vec],
    )(s_parts, d_parts)


def _mm_first(h, W):
    def kfn(h_ref, W_ref, o_ref):
        o_ref[...] = jnp.dot(h_ref[...], W_ref[...],
                             preferred_element_type=jnp.float32)

    return pl.pallas_call(
        kfn, grid=(_N // _R,),
        in_specs=[pl.BlockSpec((_R, h.shape[1]), lambda i: (i, 0)),
                  pl.BlockSpec(W.shape, lambda i: (0, 0))],
        out_specs=pl.BlockSpec((_R, W.shape[1]), lambda i: (i, 0)),
        out_shape=jax.ShapeDtypeStruct((_N, W.shape[1]), jnp.float32),
    )(h, W)


def _stage_b(parts, y, inv_sage, rs_out, bs, Wg):
    D = y.shape[1]
    D2 = Wg.shape[1]

    def kfn(p_ref, y_ref, inv_ref, rso_ref, bs_ref, W_ref, o_ref):
        agg = p_ref[0] + p_ref[1] + 2.0 * y_ref[...]
        u = jnp.maximum(agg * inv_ref[...] + bs_ref[...], 0.0)
        t = u * rso_ref[...]
        o_ref[...] = jnp.dot(t, W_ref[...], preferred_element_type=jnp.float32)

    return pl.pallas_call(
        kfn, grid=(_N // _R,),
        in_specs=[pl.BlockSpec((_NC, _R, D), lambda i: (0, i, 0)),
                  pl.BlockSpec((_R, D), lambda i: (i, 0)),
                  pl.BlockSpec((_R, 1), lambda i: (i, 0)),
                  pl.BlockSpec((_R, 1), lambda i: (i, 0)),
                  pl.BlockSpec((1, D), lambda i: (0, 0)),
                  pl.BlockSpec((D, D2), lambda i: (0, 0))],
        out_specs=pl.BlockSpec((_R, D2), lambda i: (i, 0)),
        out_shape=jax.ShapeDtypeStruct((_N, D2), jnp.float32),
    )(parts, y, inv_sage, rs_out, bs, Wg)


def _stage_c(parts, y2, rs_in, bg, W, b_out=None):
    D = y2.shape[1]
    D2 = W.shape[1]
    with_bias = b_out is not None

    def kfn(p_ref, y_ref, rsi_ref, bg_ref, W_ref, *rest):
        if with_bias:
            bo_ref, o_ref = rest
        else:
            (o_ref,) = rest
        agg = p_ref[0] + p_ref[1] + y_ref[...]
        h = jnp.maximum(agg * rsi_ref[...] + bg_ref[...], 0.0)
        o = jnp.dot(h, W_ref[...], preferred_element_type=jnp.float32)
        if with_bias:
            o = o + bo_ref[...]
        o_ref[...] = o

    in_specs = [pl.BlockSpec((_NC, _R, D), lambda i: (0, i, 0)),
                pl.BlockSpec((_R, D), lambda i: (i, 0)),
                pl.BlockSpec((_R, 1), lambda i: (i, 0)),
                pl.BlockSpec((1, D), lambda i: (0, 0)),
                pl.BlockSpec((D, D2), lambda i: (0, 0))]
    args = [parts, y2, rs_in, bg, W]
    if with_bias:
        in_specs.append(pl.BlockSpec((1, D2), lambda i: (0, 0)))
        args.append(b_out)
    return pl.pallas_call(
        kfn, grid=(_N // _R,),
        in_specs=in_specs,
        out_specs=pl.BlockSpec((_R, D2), lambda i: (i, 0)),
        out_shape=jax.ShapeDtypeStruct((_N, D2), jnp.float32),
    )(*args)


def kernel(features, edge_index, Ws0, bs0, Wg0, bg0, Ws1, bs1, Wg1, bg1,
           Ws2, bs2, Wg2, bg2, Wfc, bfc):
    src = edge_index[0].astype(jnp.int32).reshape(_NW, _NCH, _CH)
    dst = edge_index[1].astype(jnp.int32).reshape(_NW, _NCH, _CH)

    s_parts, d_parts = _deg_kernel(src, dst)
    inv_sage, rs_out, rs_in = _degprep(s_parts, d_parts)

    layers = [(bs0, Wg0, bg0, Ws1), (bs1, Wg1, bg1, Ws2),
              (bs2, Wg2, bg2, Wfc)]
    y = _mm_first(features, Ws0)
    out = None
    for s, (bs, Wg, bg, Wnext) in enumerate(layers):
        parts = _edge_scatter(y.shape[1])(y, src, dst)
        y2 = _stage_b(parts, y, inv_sage, rs_out, bs.reshape(1, -1), Wg)
        parts2 = _edge_scatter(y2.shape[1])(y2, src, dst)
        if s < 2:
            y = _stage_c(parts2, y2, rs_in, bg.reshape(1, -1), Wnext)
        else:
            out = _stage_c(parts2, y2, rs_in, bg.reshape(1, -1), Wnext,
                           bfc.reshape(1, -1))
    return out


# speedup vs baseline: 1.0237x; 1.0237x over previous
import functools

import jax
import jax.numpy as jnp
from jax import lax
from jax.experimental import pallas as pl
from jax.experimental.pallas import tpu as pltpu
from jax.experimental.pallas import tpu_sc as plsc

_N = 10000
_E = 320000
_NC = 2
_NS = 16
_NW = _NC * _NS
_NP = 10240
_RPW = _NP // _NS
_R = 1000
_NBUF = 5
_CH = 80
_NCH = _E // (_NW * _CH)


def _sc_mesh():
    return plsc.VectorSubcoreMesh(
        core_axis_name="c", subcore_axis_name="s",
        num_cores=_NC, num_subcores=_NS)


@functools.lru_cache(maxsize=None)
def _edge_scatter(D):
    ch = _CH
    nch = _NCH
    nbuf = 25 if D == 32 else 5
    ahead = 8 if D == 32 else 4
    lag = nbuf - ahead

    @functools.partial(
        pl.kernel,
        out_type=jax.ShapeDtypeStruct((_NC, _NP, D), jnp.float32),
        mesh=_sc_mesh(),
        compiler_params=pltpu.CompilerParams(use_tc_tiling_on_sc=False),
        scratch_types=[
            pltpu.VMEM((nch, ch), jnp.int32),
            pltpu.VMEM((nch, ch), jnp.int32),
            pltpu.VMEM((nbuf, ch, D), jnp.float32),
            pltpu.VMEM_SHARED((_NP, D), jnp.float32),
            pltpu.SemaphoreType.DMA,
            pltpu.SemaphoreType.DMA,
        ],
    )
    def k(y_hbm, src_hbm, dst_hbm, out_hbm, src_v, dst_v, rows_v, acc,
          gsem, ssem):
        cid = lax.axis_index("c")
        sid = lax.axis_index("s")
        wid = sid * _NC + cid
        zeros = jnp.zeros((16,), jnp.float32)

        pltpu.async_copy(src_hbm.at[wid], src_v, gsem)
        pltpu.async_copy(dst_hbm.at[wid], dst_v, gsem)

        def zero_body(i, carry):
            for j in range(D // 16):
                rows_v[ahead, i, 16 * j:16 * (j + 1)] = zeros
            return carry

        lax.fori_loop(0, ch, zero_body, 0)
        pltpu.make_async_copy(src_hbm.at[wid], src_v, gsem).wait()
        pltpu.make_async_copy(dst_hbm.at[wid], dst_v, gsem).wait()

        for t in range(ahead):
            pltpu.async_copy(y_hbm.at[src_v.at[t]], rows_v.at[t], gsem)

        for t in range(_RPW // ch):
            pltpu.async_copy(
                rows_v.at[ahead], acc.at[pl.ds(sid * _RPW + t * ch, ch)],
                ssem)
        for t in range(_RPW // ch):
            pltpu.make_async_copy(
                rows_v.at[ahead], acc.at[pl.ds(sid * _RPW + t * ch, ch)],
                ssem).wait()
        plsc.subcore_barrier()

        def outer(o, carry):
            for b in range(nbuf):
                j = nbuf * o + b
                pltpu.make_async_copy(
                    y_hbm.at[src_v.at[j]], rows_v.at[b], gsem).wait()
                pltpu.async_copy(
                    rows_v.at[b], acc.at[dst_v.at[j]], ssem, add=True)

                @pl.when(j >= lag)
                def _():
                    pltpu.make_async_copy(
                        rows_v.at[(b - lag) % nbuf], acc.at[pl.ds(0, ch)],
                        ssem).wait()

                @pl.when(j < nch - ahead)
                def _():
                    pltpu.async_copy(
                        y_hbm.at[src_v.at[j + ahead]],
                        rows_v.at[(b + ahead) % nbuf], gsem)
            return carry

        lax.fori_loop(0, nch // nbuf, outer, 0)
        for _ in range(lag):
            pltpu.make_async_copy(
                rows_v.at[0], acc.at[pl.ds(0, ch)], ssem).wait()
        plsc.subcore_barrier()
        pltpu.sync_copy(acc.at[pl.ds(sid * _RPW, _RPW)],
                        out_hbm.at[cid, pl.ds(sid * _RPW, _RPW)])

    return k


@functools.partial(
    pl.kernel,
    out_type=(jax.ShapeDtypeStruct((_NC, _NP, 16), jnp.float32),
              jax.ShapeDtypeStruct((_NC, _NP, 16), jnp.float32)),
    mesh=_sc_mesh(),
    compiler_params=pltpu.CompilerParams(use_tc_tiling_on_sc=False),
    scratch_types=[
        pltpu.VMEM((125, 80), jnp.int32),
        pltpu.VMEM((125, 80), jnp.int32),
        pltpu.VMEM((80, 16), jnp.float32),
        pltpu.VMEM_SHARED((_NP, 16), jnp.float32),
        pltpu.VMEM_SHARED((_NP, 16), jnp.float32),
        pltpu.SemaphoreType.DMA,
        pltpu.SemaphoreType.DMA,
    ],
)
def _deg_kernel(src_hbm, dst_hbm, outs_hbm, outd_hbm,
                src_v, dst_v, ones_v, acc_s, acc_d, gsem, ssem):
    cid = lax.axis_index("c")
    sid = lax.axis_index("s")
    wid = sid * _NC + cid
    zeros = jnp.zeros((16,), jnp.float32)
    ones = jnp.ones((16,), jnp.float32)

    pltpu.async_copy(src_hbm.at[wid], src_v, gsem)
    pltpu.async_copy(dst_hbm.at[wid], dst_v, gsem)

    def zfill(i, carry):
        ones_v[i, 0:16] = zeros
        return carry

    lax.fori_loop(0, 80, zfill, 0)
    for t in range(_RPW // 80):
        pltpu.async_copy(
            ones_v, acc_s.at[pl.ds(sid * _RPW + t * 80, 80)], ssem)
        pltpu.async_copy(
            ones_v, acc_d.at[pl.ds(sid * _RPW + t * 80, 80)], ssem)
    for _ in range(2 * (_RPW // 80)):
        pltpu.make_async_copy(ones_v, acc_s.at[pl.ds(0, 80)], ssem).wait()

    def ofill(i, carry):
        ones_v[i, 0:16] = ones
        return carry

    lax.fori_loop(0, 80, ofill, 0)
    pltpu.make_async_copy(src_hbm.at[wid], src_v, gsem).wait()
    pltpu.make_async_copy(dst_hbm.at[wid], dst_v, gsem).wait()
    plsc.subcore_barrier()

    def body(j, carry):
        pltpu.async_copy(ones_v, acc_s.at[src_v.at[j]], ssem, add=True)
        pltpu.async_copy(ones_v, acc_d.at[dst_v.at[j]], ssem, add=True)

        @pl.when(j >= 8)
        def _():
            pltpu.make_async_copy(
                ones_v, acc_s.at[pl.ds(0, 80)], ssem).wait()
            pltpu.make_async_copy(
                ones_v, acc_s.at[pl.ds(0, 80)], ssem).wait()

        return carry

    lax.fori_loop(0, 125, body, 0)
    for _ in range(16):
        pltpu.make_async_copy(ones_v, acc_s.at[pl.ds(0, 80)], ssem).wait()
    plsc.subcore_barrier()
    pltpu.sync_copy(acc_s.at[pl.ds(sid * _RPW, _RPW)],
                    outs_hbm.at[cid, pl.ds(sid * _RPW, _RPW)])
    pltpu.sync_copy(acc_d.at[pl.ds(sid * _RPW, _RPW)],
                    outd_hbm.at[cid, pl.ds(sid * _RPW, _RPW)])


def _degprep(s_parts, d_parts):
    def kfn(s_ref, d_ref, inv_ref, rso_ref, rsi_ref):
        s = s_ref[0, :, 0:1] + s_ref[1, :, 0:1]
        d = d_ref[0, :, 0:1] + d_ref[1, :, 0:1]
        inv_ref[...] = 1.0 / (d + 2.0)
        rso_ref[...] = lax.rsqrt(s + 1.0)
        rsi_ref[...] = lax.rsqrt(d + 1.0)

    vec = jax.ShapeDtypeStruct((_N, 1), jnp.float32)
    return pl.pallas_call(
        kfn, grid=(_N // _R,),
        in_specs=[pl.BlockSpec((_NC, _R, 16), lambda i: (0, i, 0)),
                  pl.BlockSpec((_NC, _R, 16), lambda i: (0, i, 0))],
        out_specs=[pl.BlockSpec((_R, 1), lambda i: (i, 0))] * 3,
        out_shape=[vec, vec, vec],
    )(s_parts, d_parts)


def _mm_first(h, W):
    def kfn(h_ref, W_ref, o_ref):
        o_ref[...] = jnp.dot(h_ref[...], W_ref[...],
                             preferred_element_type=jnp.float32)

    return pl.pallas_call(
        kfn, grid=(_N // _R,),
        in_specs=[pl.BlockSpec((_R, h.shape[1]), lambda i: (i, 0)),
                  pl.BlockSpec(W.shape, lambda i: (0, 0))],
        out_specs=pl.BlockSpec((_R, W.shape[1]), lambda i: (i, 0)),
        out_shape=jax.ShapeDtypeStruct((_N, W.shape[1]), jnp.float32),
    )(h, W)


def _stage_b(parts, y, inv_sage, rs_out, bs, Wg):
    D = y.shape[1]
    D2 = Wg.shape[1]

    def kfn(p_ref, y_ref, inv_ref, rso_ref, bs_ref, W_ref, o_ref):
        agg = p_ref[0] + p_ref[1] + 2.0 * y_ref[...]
        u = jnp.maximum(agg * inv_ref[...] + bs_ref[...], 0.0)
        t = u * rso_ref[...]
        o_ref[...] = jnp.dot(t, W_ref[...], preferred_element_type=jnp.float32)

    return pl.pallas_call(
        kfn, grid=(_N // _R,),
        in_specs=[pl.BlockSpec((_NC, _R, D), lambda i: (0, i, 0)),
                  pl.BlockSpec((_R, D), lambda i: (i, 0)),
                  pl.BlockSpec((_R, 1), lambda i: (i, 0)),
                  pl.BlockSpec((_R, 1), lambda i: (i, 0)),
                  pl.BlockSpec((1, D), lambda i: (0, 0)),
                  pl.BlockSpec((D, D2), lambda i: (0, 0))],
        out_specs=pl.BlockSpec((_R, D2), lambda i: (i, 0)),
        out_shape=jax.ShapeDtypeStruct((_N, D2), jnp.float32),
    )(parts, y, inv_sage, rs_out, bs, Wg)


def _stage_c(parts, y2, rs_in, bg, W, b_out=None):
    D = y2.shape[1]
    D2 = W.shape[1]
    with_bias = b_out is not None

    def kfn(p_ref, y_ref, rsi_ref, bg_ref, W_ref, *rest):
        if with_bias:
            bo_ref, o_ref = rest
        else:
            (o_ref,) = rest
        agg = p_ref[0] + p_ref[1] + y_ref[...]
        h = jnp.maximum(agg * rsi_ref[...] + bg_ref[...], 0.0)
        o = jnp.dot(h, W_ref[...], preferred_element_type=jnp.float32)
        if with_bias:
            o = o + bo_ref[...]
        o_ref[...] = o

    in_specs = [pl.BlockSpec((_NC, _R, D), lambda i: (0, i, 0)),
                pl.BlockSpec((_R, D), lambda i: (i, 0)),
                pl.BlockSpec((_R, 1), lambda i: (i, 0)),
                pl.BlockSpec((1, D), lambda i: (0, 0)),
                pl.BlockSpec((D, D2), lambda i: (0, 0))]
    args = [parts, y2, rs_in, bg, W]
    if with_bias:
        in_specs.append(pl.BlockSpec((1, D2), lambda i: (0, 0)))
        args.append(b_out)
    return pl.pallas_call(
        kfn, grid=(_N // _R,),
        in_specs=in_specs,
        out_specs=pl.BlockSpec((_R, D2), lambda i: (i, 0)),
        out_shape=jax.ShapeDtypeStruct((_N, D2), jnp.float32),
    )(*args)


def kernel(features, edge_index, Ws0, bs0, Wg0, bg0, Ws1, bs1, Wg1, bg1,
           Ws2, bs2, Wg2, bg2, Wfc, bfc):
    src = edge_index[0].astype(jnp.int32).reshape(_NW, _NCH, _CH)
    dst = edge_index[1].astype(jnp.int32).reshape(_NW, _NCH, _CH)

    s_parts, d_parts = _deg_kernel(src, dst)
    inv_sage, rs_out, rs_in = _degprep(s_parts, d_parts)

    layers = [(bs0, Wg0, bg0, Ws1), (bs1, Wg1, bg1, Ws2),
              (bs2, Wg2, bg2, Wfc)]
    y = _mm_first(features, Ws0)
    out = None
    for s, (bs, Wg, bg, Wnext) in enumerate(layers):
        parts = _edge_scatter(y.shape[1])(y, src, dst)
        y2 = _stage_b(parts, y, inv_sage, rs_out, bs.reshape(1, -1), Wg)
        parts2 = _edge_scatter(y2.shape[1])(y2, src, dst)
        if s < 2:
            y = _stage_c(parts2, y2, rs_in, bg.reshape(1, -1), Wnext)
        else:
            out = _stage_c(parts2, y2, rs_in, bg.reshape(1, -1), Wnext,
                           bfc.reshape(1, -1))
    return out
